# Initial kernel scaffold; baseline (speedup 1.0000x reference)
import jax
import jax.numpy as jnp
from jax.experimental import pallas as pl


def kernel(tensor, weight, bias, batch_index):
    raise NotImplementedError("write your pallas kernel here")


# speedup vs baseline: 3.0632x; 3.0632x over previous
import jax
import jax.numpy as jnp
from jax.experimental import pallas as pl
from jax.experimental.pallas import tpu as pltpu

_B = 512
_BLK = 256
_EPS = 1e-6


def _stats_kernel(idx_ref, x_ref, stat_ref, cnt_ref):
    pi = pl.program_id(0)
    x = x_ref[...]
    idx = idx_ref[0]
    iota = jax.lax.broadcasted_iota(jnp.int32, (_B, 1), 0)
    onehot_t = (iota == idx).astype(jnp.float32)
    xcat = jnp.concatenate([x, x * x], axis=1)
    s = jax.lax.dot_general(onehot_t, xcat, (((1,), (0,)), ((), ())),
                            preferred_element_type=jnp.float32)
    c = jnp.sum(onehot_t, axis=1, keepdims=True)

    @pl.when(pi == 0)
    def _init():
        stat_ref[...] = s
        cnt_ref[...] = c

    @pl.when(pi != 0)
    def _acc():
        stat_ref[...] += s
        cnt_ref[...] += c


def _norm_kernel(idx_ref, x_ref, stat_ref, cnt_ref, w_ref, b_ref, out_ref,
                 ss_ref):
    pi = pl.program_id(0)
    d = x_ref.shape[1]

    @pl.when(pi == 0)
    def _prep():
        stat = stat_ref[...]
        cnt = cnt_ref[...]
        inv = 1.0 / jnp.maximum(cnt, 1.0)
        mean = stat[:, :d] * inv
        var = jnp.maximum(stat[:, d:] * inv - mean * mean, 0.0)
        rstd = jax.lax.rsqrt(var + _EPS)
        scale = w_ref[...] * rstd
        shift = b_ref[...] - mean * scale
        ss_ref[...] = jnp.concatenate([scale, shift], axis=1)

    x = x_ref[...]
    idx = idx_ref[...]
    iota = jax.lax.broadcasted_iota(jnp.int32, (1, _B), 1)
    onehot = (idx == iota).astype(jnp.float32)
    g = jax.lax.dot_general(onehot, ss_ref[...], (((1,), (0,)), ((), ())),
                            preferred_element_type=jnp.float32)
    out_ref[...] = x * g[:, :d] + g[:, d:]


def kernel(tensor, weight, bias, batch_index):
    n, d = tensor.shape
    idx = batch_index.astype(jnp.int32)
    nblk = pl.cdiv(n, _BLK)
    npad = nblk * _BLK
    pad = npad - n
    x = jnp.pad(tensor, ((0, pad), (0, 0)))
    idx_p = jnp.pad(idx, (0, pad), constant_values=_B)
    idx3 = idx_p.reshape(nblk, 1, _BLK)
    idx2 = idx_p.reshape(npad, 1)
    w2 = weight.reshape(1, d)
    b2 = bias.reshape(1, d)

    stat, cnt = pl.pallas_call(
        _stats_kernel,
        grid=(nblk,),
        in_specs=[
            pl.BlockSpec((1, 1, _BLK), lambda i: (i, 0, 0)),
            pl.BlockSpec((_BLK, d), lambda i: (i, 0)),
        ],
        out_specs=[
            pl.BlockSpec((_B, 2 * d), lambda i: (0, 0)),
            pl.BlockSpec((_B, 1), lambda i: (0, 0)),
        ],
        out_shape=[
            jax.ShapeDtypeStruct((_B, 2 * d), jnp.float32),
            jax.ShapeDtypeStruct((_B, 1), jnp.float32),
        ],
    )(idx3, x)

    out = pl.pallas_call(
        _norm_kernel,
        grid=(nblk,),
        in_specs=[
            pl.BlockSpec((_BLK, 1), lambda i: (i, 0)),
            pl.BlockSpec((_BLK, d), lambda i: (i, 0)),
            pl.BlockSpec((_B, 2 * d), lambda i: (0, 0)),
            pl.BlockSpec((_B, 1), lambda i: (0, 0)),
            pl.BlockSpec((1, d), lambda i: (0, 0)),
            pl.BlockSpec((1, d), lambda i: (0, 0)),
        ],
        out_specs=pl.BlockSpec((_BLK, d), lambda i: (i, 0)),
        out_shape=jax.ShapeDtypeStruct((npad, d), jnp.float32),
        scratch_shapes=[pltpu.VMEM((_B, 2 * d), jnp.float32)],
    )(idx2, x, stat, cnt, w2, b2)

    return out[:n]


# speedup vs baseline: 3.2373x; 1.0568x over previous
import jax
import jax.numpy as jnp
from jax.experimental import pallas as pl
from jax.experimental.pallas import tpu as pltpu

_B = 512
_BLK = 256
_W = 32
_EPS = 1e-6


def _stats_kernel(idx_ref, x_ref, stat_ref, cnt_ref):
    pi = pl.program_id(0)

    @pl.when(pi == 0)
    def _zero():
        stat_ref[...] = jnp.zeros_like(stat_ref)
        cnt_ref[...] = jnp.zeros_like(cnt_ref)

    x = x_ref[...]
    idx = idx_ref[0]
    lo = idx_ref[0, 0, 0]
    hi = idx_ref[0, 0, _BLK - 1]
    xcat = jnp.concatenate([x, x * x], axis=1)

    narrow = (hi - lo) < (_W - 8)
    lo_a = jnp.minimum((lo // 8) * 8, _B - _W)

    @pl.when(narrow)
    def _narrow():
        iota = jax.lax.broadcasted_iota(jnp.int32, (_W, 1), 0) + lo_a
        oh = (iota == idx).astype(jnp.float32)
        s = jax.lax.dot_general(oh, xcat, (((1,), (0,)), ((), ())),
                                preferred_element_type=jnp.float32)
        stat_ref[pl.ds(lo_a, _W), :] += s
        cnt_ref[pl.ds(lo_a, _W), :] += jnp.sum(oh, axis=1, keepdims=True)

    @pl.when(jnp.logical_not(narrow))
    def _wide():
        iota = jax.lax.broadcasted_iota(jnp.int32, (_B, 1), 0)
        oh = (iota == idx).astype(jnp.float32)
        s = jax.lax.dot_general(oh, xcat, (((1,), (0,)), ((), ())),
                                preferred_element_type=jnp.float32)
        stat_ref[...] += s
        cnt_ref[...] += jnp.sum(oh, axis=1, keepdims=True)


def _norm_kernel(idx_ref, x_ref, stat_ref, cnt_ref, w_ref, b_ref, out_ref,
                 ss_ref):
    pi = pl.program_id(0)
    d = x_ref.shape[1]

    @pl.when(pi == 0)
    def _prep():
        stat = stat_ref[...]
        cnt = cnt_ref[...]
        inv = 1.0 / jnp.maximum(cnt, 1.0)
        mean = stat[:, :d] * inv
        var = jnp.maximum(stat[:, d:] * inv - mean * mean, 0.0)
        rstd = jax.lax.rsqrt(var + _EPS)
        scale = w_ref[...] * rstd
        shift = b_ref[...] - mean * scale
        ss_ref[...] = jnp.concatenate([scale, shift], axis=1)

    x = x_ref[...]
    idx = idx_ref[...]
    lo = idx_ref[0, 0]
    hi = idx_ref[_BLK - 1, 0]
    narrow = (hi - lo) < (_W - 8)
    lo_a = jnp.minimum((lo // 8) * 8, _B - _W)

    @pl.when(narrow)
    def _narrow():
        iota = jax.lax.broadcasted_iota(jnp.int32, (1, _W), 1) + lo_a
        oh = (idx == iota).astype(jnp.float32)
        g = jax.lax.dot_general(oh, ss_ref[pl.ds(lo_a, _W), :],
                                (((1,), (0,)), ((), ())),
                                preferred_element_type=jnp.float32)
        out_ref[...] = x * g[:, :d] + g[:, d:]

    @pl.when(jnp.logical_not(narrow))
    def _wide():
        iota = jax.lax.broadcasted_iota(jnp.int32, (1, _B), 1)
        oh = (idx == iota).astype(jnp.float32)
        g = jax.lax.dot_general(oh, ss_ref[...], (((1,), (0,)), ((), ())),
                                preferred_element_type=jnp.float32)
        out_ref[...] = x * g[:, :d] + g[:, d:]


def kernel(tensor, weight, bias, batch_index):
    n, d = tensor.shape
    idx = batch_index.astype(jnp.int32)
    nblk = pl.cdiv(n, _BLK)
    npad = nblk * _BLK
    pad = npad - n
    x = jnp.pad(tensor, ((0, pad), (0, 0)))
    idx_p = jnp.pad(idx, (0, pad), constant_values=_B)
    idx3 = idx_p.reshape(nblk, 1, _BLK)
    idx2 = idx_p.reshape(npad, 1)
    w2 = weight.reshape(1, d)
    b2 = bias.reshape(1, d)

    stat, cnt = pl.pallas_call(
        _stats_kernel,
        grid=(nblk,),
        in_specs=[
            pl.BlockSpec((1, 1, _BLK), lambda i: (i, 0, 0)),
            pl.BlockSpec((_BLK, d), lambda i: (i, 0)),
        ],
        out_specs=[
            pl.BlockSpec((_B, 2 * d), lambda i: (0, 0)),
            pl.BlockSpec((_B, 1), lambda i: (0, 0)),
        ],
        out_shape=[
            jax.ShapeDtypeStruct((_B, 2 * d), jnp.float32),
            jax.ShapeDtypeStruct((_B, 1), jnp.float32),
        ],
    )(idx3, x)

    out = pl.pallas_call(
        _norm_kernel,
        grid=(nblk,),
        in_specs=[
            pl.BlockSpec((_BLK, 1), lambda i: (i, 0)),
            pl.BlockSpec((_BLK, d), lambda i: (i, 0)),
            pl.BlockSpec((_B, 2 * d), lambda i: (0, 0)),
            pl.BlockSpec((_B, 1), lambda i: (0, 0)),
            pl.BlockSpec((1, d), lambda i: (0, 0)),
            pl.BlockSpec((1, d), lambda i: (0, 0)),
        ],
        out_specs=pl.BlockSpec((_BLK, d), lambda i: (i, 0)),
        out_shape=jax.ShapeDtypeStruct((npad, d), jnp.float32),
        scratch_shapes=[pltpu.VMEM((_B, 2 * d), jnp.float32)],
    )(idx2, x, stat, cnt, w2, b2)

    return out[:n]


# speedup vs baseline: 4.6810x; 1.4459x over previous
import jax
import jax.numpy as jnp
from jax.experimental import pallas as pl
from jax.experimental.pallas import tpu as pltpu

_B = 512
_BLK = 512
_W = 32
_EPS = 1e-6


def _stats_kernel(idx_ref, x_ref, sum_ref, sq_ref, cnt_ref):
    pi = pl.program_id(0)

    @pl.when(pi == 0)
    def _zero():
        sum_ref[...] = jnp.zeros_like(sum_ref)
        sq_ref[...] = jnp.zeros_like(sq_ref)
        cnt_ref[...] = jnp.zeros_like(cnt_ref)

    x = x_ref[...]
    idx = idx_ref[0]
    lo = idx_ref[0, 0, 0]
    hi = idx_ref[0, 0, _BLK - 1]

    narrow = (hi - lo) < (_W - 8)
    lo_a = jnp.minimum((lo // 8) * 8, _B - _W)

    @pl.when(narrow)
    def _narrow():
        iota = jax.lax.broadcasted_iota(jnp.int32, (_W, 1), 0) + lo_a
        oh = (iota == idx).astype(jnp.float32)
        s = jax.lax.dot_general(oh, x, (((1,), (0,)), ((), ())),
                                preferred_element_type=jnp.float32)
        q = jax.lax.dot_general(oh, x * x, (((1,), (0,)), ((), ())),
                                preferred_element_type=jnp.float32)
        sum_ref[pl.ds(lo_a, _W), :] += s
        sq_ref[pl.ds(lo_a, _W), :] += q
        cnt_ref[pl.ds(lo_a, _W), :] += jnp.sum(oh, axis=1, keepdims=True)

    @pl.when(jnp.logical_not(narrow))
    def _wide():
        iota = jax.lax.broadcasted_iota(jnp.int32, (_B, 1), 0)
        oh = (iota == idx).astype(jnp.float32)
        s = jax.lax.dot_general(oh, x, (((1,), (0,)), ((), ())),
                                preferred_element_type=jnp.float32)
        q = jax.lax.dot_general(oh, x * x, (((1,), (0,)), ((), ())),
                                preferred_element_type=jnp.float32)
        sum_ref[...] += s
        sq_ref[...] += q
        cnt_ref[...] += jnp.sum(oh, axis=1, keepdims=True)


def _norm_kernel(idx_ref, x_ref, sum_ref, sq_ref, cnt_ref, w_ref, b_ref,
                 out_ref, sc_ref, sh_ref):
    pi = pl.program_id(0)

    @pl.when(pi == 0)
    def _prep():
        cnt = cnt_ref[...]
        inv = 1.0 / jnp.maximum(cnt, 1.0)
        mean = sum_ref[...] * inv
        var = jnp.maximum(sq_ref[...] * inv - mean * mean, 0.0)
        rstd = jax.lax.rsqrt(var + _EPS)
        scale = w_ref[...] * rstd
        sc_ref[...] = scale
        sh_ref[...] = b_ref[...] - mean * scale

    x = x_ref[...]
    idx = idx_ref[...]
    lo = idx_ref[0, 0]
    hi = idx_ref[_BLK - 1, 0]
    narrow = (hi - lo) < (_W - 8)
    lo_a = jnp.minimum((lo // 8) * 8, _B - _W)

    @pl.when(narrow)
    def _narrow():
        iota = jax.lax.broadcasted_iota(jnp.int32, (1, _W), 1) + lo_a
        oh = (idx == iota).astype(jnp.float32)
        gs = jax.lax.dot_general(oh, sc_ref[pl.ds(lo_a, _W), :],
                                 (((1,), (0,)), ((), ())),
                                 preferred_element_type=jnp.float32)
        gt = jax.lax.dot_general(oh, sh_ref[pl.ds(lo_a, _W), :],
                                 (((1,), (0,)), ((), ())),
                                 preferred_element_type=jnp.float32)
        out_ref[...] = x * gs + gt

    @pl.when(jnp.logical_not(narrow))
    def _wide():
        iota = jax.lax.broadcasted_iota(jnp.int32, (1, _B), 1)
        oh = (idx == iota).astype(jnp.float32)
        gs = jax.lax.dot_general(oh, sc_ref[...], (((1,), (0,)), ((), ())),
                                 preferred_element_type=jnp.float32)
        gt = jax.lax.dot_general(oh, sh_ref[...], (((1,), (0,)), ((), ())),
                                 preferred_element_type=jnp.float32)
        out_ref[...] = x * gs + gt


def kernel(tensor, weight, bias, batch_index):
    n, d = tensor.shape
    idx = batch_index.astype(jnp.int32)
    nblk = pl.cdiv(n, _BLK)
    npad = nblk * _BLK
    pad = npad - n
    x = jnp.pad(tensor, ((0, pad), (0, 0)))
    idx_p = jnp.pad(idx, (0, pad), constant_values=_B)
    idx3 = idx_p.reshape(nblk, 1, _BLK)
    idx2 = idx_p.reshape(npad, 1)
    w2 = weight.reshape(1, d)
    b2 = bias.reshape(1, d)

    sums, sqs, cnt = pl.pallas_call(
        _stats_kernel,
        grid=(nblk,),
        in_specs=[
            pl.BlockSpec((1, 1, _BLK), lambda i: (i, 0, 0)),
            pl.BlockSpec((_BLK, d), lambda i: (i, 0)),
        ],
        out_specs=[
            pl.BlockSpec((_B, d), lambda i: (0, 0)),
            pl.BlockSpec((_B, d), lambda i: (0, 0)),
            pl.BlockSpec((_B, 1), lambda i: (0, 0)),
        ],
        out_shape=[
            jax.ShapeDtypeStruct((_B, d), jnp.float32),
            jax.ShapeDtypeStruct((_B, d), jnp.float32),
            jax.ShapeDtypeStruct((_B, 1), jnp.float32),
        ],
    )(idx3, x)

    out = pl.pallas_call(
        _norm_kernel,
        grid=(nblk,),
        in_specs=[
            pl.BlockSpec((_BLK, 1), lambda i: (i, 0)),
            pl.BlockSpec((_BLK, d), lambda i: (i, 0)),
            pl.BlockSpec((_B, d), lambda i: (0, 0)),
            pl.BlockSpec((_B, d), lambda i: (0, 0)),
            pl.BlockSpec((_B, 1), lambda i: (0, 0)),
            pl.BlockSpec((1, d), lambda i: (0, 0)),
            pl.BlockSpec((1, d), lambda i: (0, 0)),
        ],
        out_specs=pl.BlockSpec((_BLK, d), lambda i: (i, 0)),
        out_shape=jax.ShapeDtypeStruct((npad, d), jnp.float32),
        scratch_shapes=[pltpu.VMEM((_B, d), jnp.float32),
                        pltpu.VMEM((_B, d), jnp.float32)],
    )(idx2, x, sums, sqs, cnt, w2, b2)

    return out[:n]
